# baseline (speedup 1.0000x reference)
import functools

import jax
import jax.numpy as jnp
from jax.experimental import pallas as pl
from jax.experimental.pallas import tpu as pltpu
from jax.experimental.pallas import tpu_sc as plsc

LATENT_DIM = 128

_mesh = plsc.ScalarSubcoreMesh(axis_name="c", num_cores=1)


@functools.partial(
    pl.kernel,
    mesh=_mesh,
    out_type=jax.ShapeDtypeStruct((1, LATENT_DIM), jnp.float32),
)
def _sc_row_copy(w_hbm, out_hbm):
    pltpu.sync_copy(w_hbm, out_hbm)


def kernel(user_emb_weight):
    return _sc_row_copy(user_emb_weight)
